# baseline (speedup 1.0000x reference)
import functools

import jax
import jax.numpy as jnp
from jax import lax
from jax.experimental import pallas as pl
from jax.experimental.pallas import tpu as pltpu
from jax.experimental.pallas import tpu_sc as plsc

D_MODEL = 64
D_PAD = 128
SCALE = 8.0

NUM_CORES = 2
NUM_SUBCORES = 16
NUM_WORKERS = NUM_CORES * NUM_SUBCORES

B_TOTAL = 4096 * 200
ROWS_PER_WORKER = B_TOTAL // NUM_WORKERS
CHUNK = 400
NUM_CHUNKS = ROWS_PER_WORKER // CHUNK
NBUF = 2
LANES = 16


VOCAB = 1000000
PAD_CHUNK = 200
PAD_NCHUNK = VOCAB // PAD_CHUNK
PAD_KMAX = 158


def _pad_body(lut_hbm, out_hbm, n0, n1, w0, w1, is0, is1, os0, os1):
    wid = lax.axis_index("s") * NUM_CORES + lax.axis_index("c")
    narrow = (n0, n1)
    wide = (w0, w1)
    isem = (is0, is1)
    osem = (os0, os1)

    def chunk_of(k):
        return wid + NUM_WORKERS * k

    def start_in(c, b):
        pltpu.async_copy(lut_hbm.at[pl.ds(c * PAD_CHUNK, PAD_CHUNK)],
                         narrow[b], isem[b])

    def wait_in(c, b):
        pltpu.make_async_copy(lut_hbm.at[pl.ds(c * PAD_CHUNK, PAD_CHUNK)],
                              narrow[b], isem[b]).wait()

    def start_out(c, b):
        pltpu.async_copy(wide[b], out_hbm.at[pl.ds(c * PAD_CHUNK, PAD_CHUNK)],
                         osem[b])

    def wait_out(c, b):
        pltpu.make_async_copy(wide[b],
                              out_hbm.at[pl.ds(c * PAD_CHUNK, PAD_CHUNK)],
                              osem[b]).wait()

    def widen(b):
        src = narrow[b]
        dst = wide[b]

        @plsc.parallel_loop(0, PAD_CHUNK, step=1, unroll=8)
        def _(r):
            for j in range(D_MODEL // LANES):
                sl = pl.ds(j * LANES, LANES)
                dst[r, sl] = src[r, sl]

    @pl.when(chunk_of(0) < PAD_NCHUNK)
    def _():
        start_in(chunk_of(0), 0)

    @pl.when(chunk_of(1) < PAD_NCHUNK)
    def _():
        start_in(chunk_of(1), 1)

    @pl.loop(0, PAD_KMAX, step=NBUF)
    def _(k):
        for b in range(NBUF):
            kk = k + b

            @pl.when(chunk_of(kk) < PAD_NCHUNK)
            def _():
                c = chunk_of(kk)
                wait_in(c, b)

                @pl.when(kk >= NBUF)
                def _():
                    wait_out(chunk_of(kk - NBUF), b)

                widen(b)

                @pl.when(chunk_of(kk + NBUF) < PAD_NCHUNK)
                def _():
                    start_in(chunk_of(kk + NBUF), b)

                start_out(c, b)

    for b in range(NBUF):
        last = PAD_KMAX - NBUF + b

        @pl.when(chunk_of(last) < PAD_NCHUNK)
        def _():
            wait_out(chunk_of(last), b)


_padder = functools.partial(
    pl.kernel,
    out_type=jax.ShapeDtypeStruct((VOCAB, D_PAD), jnp.float32),
    mesh=plsc.VectorSubcoreMesh(
        core_axis_name="c",
        subcore_axis_name="s",
        num_cores=NUM_CORES,
        num_subcores=NUM_SUBCORES,
    ),
    scratch_types=[
        pltpu.VMEM((PAD_CHUNK, D_MODEL), jnp.float32),
        pltpu.VMEM((PAD_CHUNK, D_MODEL), jnp.float32),
        pltpu.VMEM((PAD_CHUNK, D_PAD), jnp.float32),
        pltpu.VMEM((PAD_CHUNK, D_PAD), jnp.float32),
        pltpu.SemaphoreType.DMA,
        pltpu.SemaphoreType.DMA,
        pltpu.SemaphoreType.DMA,
        pltpu.SemaphoreType.DMA,
    ],
    compiler_params=pltpu.CompilerParams(use_tc_tiling_on_sc=True),
)(_pad_body)


def _scale_chunk(src, dst):
    @plsc.parallel_loop(0, CHUNK, step=1, unroll=8)
    def _(r):
        for j in range(D_MODEL // LANES):
            sl = pl.ds(j * LANES, LANES)
            dst[r, sl] = src[r, sl] * SCALE


def _emb_body(x_hbm, lut_hbm, out_hbm, idx_v,
              rows0, rows1, stage0, stage1, gs0, gs1, os0, os1):
    wid = lax.axis_index("s") * NUM_CORES + lax.axis_index("c")
    base = wid * ROWS_PER_WORKER
    rows = (rows0, rows1)
    stage = (stage0, stage1)
    gsem = (gs0, gs1)
    osem = (os0, os1)

    pltpu.sync_copy(x_hbm.at[pl.ds(base, ROWS_PER_WORKER)], idx_v)

    @plsc.parallel_loop(0, ROWS_PER_WORKER // LANES, step=1, unroll=8)
    def _(i):
        sl = pl.ds(i * LANES, LANES)
        idx_v[sl] = idx_v[sl] * 2

    def start_gather(g, b):
        pltpu.async_copy(
            lut_hbm.at[idx_v.at[pl.ds(g * CHUNK, CHUNK)]], rows[b], gsem[b])

    def wait_gather(g, b):
        pltpu.make_async_copy(
            lut_hbm.at[idx_v.at[pl.ds(g * CHUNK, CHUNK)]], rows[b],
            gsem[b]).wait()

    def start_out(g, b):
        pltpu.async_copy(
            stage[b],
            out_hbm.at[pl.ds(base + g * CHUNK, CHUNK), pl.ds(0, D_MODEL)],
            osem[b])

    def wait_out(g, b):
        pltpu.make_async_copy(
            stage[b],
            out_hbm.at[pl.ds(base + g * CHUNK, CHUNK), pl.ds(0, D_MODEL)],
            osem[b]).wait()

    start_gather(0, 0)
    start_gather(1, 1)

    @pl.loop(0, NUM_CHUNKS, step=NBUF)
    def _(g):
        for b in range(NBUF):
            gg = g + b
            wait_gather(gg, b)

            @pl.when(gg >= NBUF)
            def _():
                wait_out(gg - NBUF, b)

            _scale_chunk(rows[b], stage[b])

            @pl.when(gg + NBUF < NUM_CHUNKS)
            def _():
                start_gather(gg + NBUF, b)

            start_out(gg, b)

    for b in range(NBUF):
        wait_out(NUM_CHUNKS - NBUF + b, b)


_emb = functools.partial(
    pl.kernel,
    out_type=jax.ShapeDtypeStruct((B_TOTAL, D_PAD), jnp.float32),
    mesh=plsc.VectorSubcoreMesh(
        core_axis_name="c",
        subcore_axis_name="s",
        num_cores=NUM_CORES,
        num_subcores=NUM_SUBCORES,
    ),
    scratch_types=[
        pltpu.VMEM((ROWS_PER_WORKER,), jnp.int32),
        pltpu.VMEM((CHUNK, D_MODEL), jnp.float32),
        pltpu.VMEM((CHUNK, D_MODEL), jnp.float32),
        pltpu.VMEM((CHUNK, D_MODEL), jnp.float32),
        pltpu.VMEM((CHUNK, D_MODEL), jnp.float32),
        pltpu.SemaphoreType.DMA,
        pltpu.SemaphoreType.DMA,
        pltpu.SemaphoreType.DMA,
        pltpu.SemaphoreType.DMA,
    ],
    compiler_params=pltpu.CompilerParams(use_tc_tiling_on_sc=False,
                                         needs_layout_passes=False),
)(_emb_body)


@jax.jit
def kernel(x, lut):
    lut_padded = _padder(lut)
    lut_lin = lut_padded.reshape(2 * 1000000, D_MODEL)
    flat = _emb(x.reshape(-1), lut_lin)
    return flat[:, :D_MODEL].reshape(x.shape + (D_MODEL,))


# speedup vs baseline: 1.1889x; 1.1889x over previous
import functools

import jax
import jax.numpy as jnp
from jax import lax
from jax.experimental import pallas as pl
from jax.experimental.pallas import tpu as pltpu
from jax.experimental.pallas import tpu_sc as plsc

D_MODEL = 64
D_PAD = 128
SCALE = 8.0

NUM_CORES = 2
NUM_SUBCORES = 16
NUM_WORKERS = NUM_CORES * NUM_SUBCORES

B_TOTAL = 4096 * 200
ROWS_PER_WORKER = B_TOTAL // NUM_WORKERS
CHUNK = 400
NUM_CHUNKS = ROWS_PER_WORKER // CHUNK
NBUF = 2
LANES = 16


def _scale_chunk(src, dst):
    @plsc.parallel_loop(0, CHUNK, step=1, unroll=8)
    def _(r):
        for j in range(D_MODEL // LANES):
            sl = pl.ds(j * LANES, LANES)
            dst[r, sl] = src[r, sl] * SCALE


def _emb_body(x_hbm, lut_hbm, out_hbm, idx_v,
              rows0, rows1, stage0, stage1, gs0, gs1, os0, os1):
    wid = lax.axis_index("s") * NUM_CORES + lax.axis_index("c")
    base = wid * ROWS_PER_WORKER
    rows = (rows0, rows1)
    stage = (stage0, stage1)
    gsem = (gs0, gs1)
    osem = (os0, os1)

    pltpu.sync_copy(x_hbm.at[pl.ds(base, ROWS_PER_WORKER)], idx_v)

    @plsc.parallel_loop(0, ROWS_PER_WORKER // LANES, step=1, unroll=8)
    def _(i):
        sl = pl.ds(i * LANES, LANES)
        idx_v[sl] = idx_v[sl] * 2

    def start_gather(g, b):
        pltpu.async_copy(
            lut_hbm.at[idx_v.at[pl.ds(g * CHUNK, CHUNK)]], rows[b], gsem[b])

    def wait_gather(g, b):
        pltpu.make_async_copy(
            lut_hbm.at[idx_v.at[pl.ds(g * CHUNK, CHUNK)]], rows[b],
            gsem[b]).wait()

    def start_out(g, b):
        pltpu.async_copy(
            stage[b],
            out_hbm.at[pl.ds(base + g * CHUNK, CHUNK), pl.ds(0, D_MODEL)],
            osem[b])

    def wait_out(g, b):
        pltpu.make_async_copy(
            stage[b],
            out_hbm.at[pl.ds(base + g * CHUNK, CHUNK), pl.ds(0, D_MODEL)],
            osem[b]).wait()

    start_gather(0, 0)
    start_gather(1, 1)

    @pl.loop(0, NUM_CHUNKS, step=NBUF)
    def _(g):
        for b in range(NBUF):
            gg = g + b
            wait_gather(gg, b)

            @pl.when(gg >= NBUF)
            def _():
                wait_out(gg - NBUF, b)

            _scale_chunk(rows[b], stage[b])

            @pl.when(gg + NBUF < NUM_CHUNKS)
            def _():
                start_gather(gg + NBUF, b)

            start_out(gg, b)

    for b in range(NBUF):
        wait_out(NUM_CHUNKS - NBUF + b, b)


_emb = functools.partial(
    pl.kernel,
    out_type=jax.ShapeDtypeStruct((B_TOTAL, D_PAD), jnp.float32),
    mesh=plsc.VectorSubcoreMesh(
        core_axis_name="c",
        subcore_axis_name="s",
        num_cores=NUM_CORES,
        num_subcores=NUM_SUBCORES,
    ),
    scratch_types=[
        pltpu.VMEM((ROWS_PER_WORKER,), jnp.int32),
        pltpu.VMEM((CHUNK, D_MODEL), jnp.float32),
        pltpu.VMEM((CHUNK, D_MODEL), jnp.float32),
        pltpu.VMEM((CHUNK, D_MODEL), jnp.float32),
        pltpu.VMEM((CHUNK, D_MODEL), jnp.float32),
        pltpu.SemaphoreType.DMA,
        pltpu.SemaphoreType.DMA,
        pltpu.SemaphoreType.DMA,
        pltpu.SemaphoreType.DMA,
    ],
    compiler_params=pltpu.CompilerParams(use_tc_tiling_on_sc=False,
                                         needs_layout_passes=False),
)(_emb_body)


@jax.jit
def kernel(x, lut):
    lut_padded = jnp.pad(lut, ((0, 0), (0, D_PAD - D_MODEL)))
    lut_lin = lut_padded.reshape(2 * 1000000, D_MODEL)
    flat = _emb(x.reshape(-1), lut_lin)
    return flat[:, :D_MODEL].reshape(x.shape + (D_MODEL,))
